# baseline (speedup 1.0000x reference)
import functools

import jax
import jax.numpy as jnp
from jax import lax
from jax.experimental import pallas as pl
from jax.experimental.pallas import tpu as pltpu
from jax.experimental.pallas import tpu_sc as plsc

NC = 2
NS = 16
NW = NC * NS

B = 16384
D = 64
K = 20
J = K + 1
BPW = B // NW
C = 32
NCH = BPW // C
SCOL = 32
VROWS = C * J


def _sc_body(gt_hbm, gidx_hbm, pu_hbm, pv_hbm, u_hbm, v_hbm, out_hbm,
             gt_v, gidx_v, pu_v, pv_v, urows_v, vrows_v, scores_v, sem):
    wid = lax.axis_index("s") * NC + lax.axis_index("c")
    base = wid * BPW
    pltpu.sync_copy(gt_hbm.at[pl.ds(base, BPW)], gt_v)
    pltpu.sync_copy(pu_hbm.at[pl.ds(base, BPW)], pu_v)
    pltpu.sync_copy(gidx_hbm.at[pl.ds(base * J, BPW * J)], gidx_v)
    pltpu.sync_copy(pv_hbm.at[pl.ds(base * J, BPW * J)], pv_v)

    for c in range(NCH):
        cps = [pltpu.async_copy(u_hbm.at[gt_v.at[pl.ds(c * C, C)]],
                                urows_v, sem)]
        roff = c * VROWS
        nfull, tail = VROWS // 128, VROWS % 128
        for i in range(nfull):
            cps.append(pltpu.async_copy(
                v_hbm.at[gidx_v.at[pl.ds(roff + i * 128, 128)]],
                vrows_v.at[pl.ds(i * 128, 128)], sem))
        if tail:
            cps.append(pltpu.async_copy(
                v_hbm.at[gidx_v.at[pl.ds(roff + nfull * 128, tail)]],
                vrows_v.at[pl.ds(nfull * 128, tail)], sem))
        for cp in cps:
            cp.wait()

        lanes = lax.iota(jnp.int32, 16)
        m15 = lanes == 15
        zero = lanes * 0

        def bbody(b, carry):
            pu_b = plsc.load_gather(pu_v, [zero + (c * C + b)])
            u = [jnp.where(pu_b == 1,
                           urows_v[b, pl.ds(64 + 16 * q, 16)],
                           urows_v[b, pl.ds(16 * q, 16)])
                 for q in range(4)]
            r0 = b * J
            bfull = zero + b
            f0 = zero + (c * C + b) * J
            for j in range(J):
                pv_bj = plsc.load_gather(pv_v, [f0 + j])
                p = zero.astype(jnp.float32)
                for q in range(4):
                    rsel = jnp.where(pv_bj == 1,
                                     vrows_v[r0 + j, pl.ds(64 + 16 * q, 16)],
                                     vrows_v[r0 + j, pl.ds(16 * q, 16)])
                    p = p + u[q] * rsel
                cs = plsc.cumsum(p)
                plsc.store_scatter(scores_v, [bfull, zero + j], cs,
                                   mask=m15)
            return carry

        lax.fori_loop(0, C, bbody, 0)
        pltpu.sync_copy(scores_v, out_hbm.at[pl.ds(base + c * C, C), :])


_sc_scores = functools.partial(
    pl.kernel, _sc_body,
    out_type=jax.ShapeDtypeStruct((B, SCOL), jnp.float32),
    mesh=plsc.VectorSubcoreMesh(core_axis_name="c", subcore_axis_name="s",
                                num_cores=NC, num_subcores=NS),
    compiler_params=pltpu.CompilerParams(needs_layout_passes=False),
    scratch_types=[
        pltpu.VMEM((BPW,), jnp.int32),
        pltpu.VMEM((BPW * J,), jnp.int32),
        pltpu.VMEM((BPW,), jnp.int32),
        pltpu.VMEM((BPW * J,), jnp.int32),
        pltpu.VMEM((C, 2 * D), jnp.float32),
        pltpu.VMEM((VROWS, 2 * D), jnp.float32),
        pltpu.VMEM((C, SCOL), jnp.float32),
        pltpu.SemaphoreType.DMA,
    ],
)()


EMB = 1000000
EMBP = 1000064
TJ = 256
NBF = EMB // TJ
GMAX = NBF // NW + 1


def _sc_repack_body(ut_hbm, vt_hbm, ud_hbm, vd_hbm, in_v, out_v):
    wid = lax.axis_index("s") * NC + lax.axis_index("c")
    lanes = lax.iota(jnp.int32, 16)
    zero = lanes * 0
    rowidx = [lanes + 16 * q for q in range(4)]

    def transpose_block(src_hbm, dst_hbm, j0, length):
        j0 = pl.multiple_of(j0, TJ)
        pltpu.sync_copy(src_hbm.at[:, pl.ds(j0, length)],
                        in_v.at[:, pl.ds(0, length)])

        def pbody(p, carry):
            for half in range(2):
                col = zero + (2 * p + half)
                for q in range(4):
                    out_v[p, pl.ds(64 * half + 16 * q, 16)] = (
                        plsc.load_gather(in_v, [rowidx[q], col]))
            return carry

        lax.fori_loop(0, length // 2, pbody, 0)
        pltpu.sync_copy(out_v.at[pl.ds(0, length // 2), :],
                        dst_hbm.at[pl.ds(pl.multiple_of(j0 // 2, TJ // 2),
                                         length // 2), :])

    def gbody(g, carry):
        blk = g * NW + wid

        @pl.when(blk < NBF)
        def _():
            transpose_block(ut_hbm, ud_hbm, blk * TJ, TJ)
            transpose_block(vt_hbm, vd_hbm, blk * TJ, TJ)

        return carry

    lax.fori_loop(0, GMAX, gbody, 0)

    @pl.when(wid == NBF % NW)
    def _():
        j0t = NBF * TJ + wid * 0
        transpose_block(ut_hbm, ud_hbm, j0t, 128)
        transpose_block(vt_hbm, vd_hbm, j0t, 128)


_sc_repack = functools.partial(
    pl.kernel, _sc_repack_body,
    out_type=(jax.ShapeDtypeStruct((EMBP // 2, 2 * D), jnp.float32),
              jax.ShapeDtypeStruct((EMBP // 2, 2 * D), jnp.float32)),
    mesh=plsc.VectorSubcoreMesh(core_axis_name="c", subcore_axis_name="s",
                                num_cores=NC, num_subcores=NS),
    compiler_params=pltpu.CompilerParams(needs_layout_passes=False),
    scratch_types=[
        pltpu.VMEM((D, TJ), jnp.float32),
        pltpu.VMEM((TJ // 2, 2 * D), jnp.float32),
    ],
)()


def _tc_finish_body(s_ref, o_ref):
    x = s_ref[...]
    col = lax.broadcasted_iota(jnp.int32, x.shape, 1)
    y = jnp.where(col == 0, -x, x)
    sp = jnp.maximum(y, 0.0) + jnp.log1p(jnp.exp(-jnp.abs(y)))
    sp = jnp.where(col < J, sp, 0.0)
    o_ref[...] = jnp.sum(sp, axis=1)


_TCR = 2048


def _tc_finish(scores):
    return pl.pallas_call(
        _tc_finish_body,
        grid=(B // _TCR,),
        in_specs=[pl.BlockSpec((_TCR, SCOL), lambda i: (i, 0))],
        out_specs=pl.BlockSpec((_TCR,), lambda i: (i,)),
        out_shape=jax.ShapeDtypeStruct((B,), jnp.float32),
    )(scores)


def kernel(target, context, neg, u_weight, v_weight):
    tgt = target.astype(jnp.int32)
    cat = jnp.concatenate(
        [context.astype(jnp.int32)[:, None], neg.astype(jnp.int32)],
        axis=1).reshape(-1)
    u2, v2 = _sc_repack(u_weight.T, v_weight.T)
    scores = _sc_scores(tgt >> 1, cat >> 1, tgt & 1, cat & 1, u2, v2)
    return _tc_finish(scores)


# speedup vs baseline: 1.6611x; 1.6611x over previous
import functools

import jax
import jax.numpy as jnp
from jax import lax
from jax.experimental import pallas as pl
from jax.experimental.pallas import tpu as pltpu
from jax.experimental.pallas import tpu_sc as plsc

NC = 2
NS = 16
NW = NC * NS

B = 16384
D = 64
K = 20
J = K + 1
BPW = B // NW
C = 32
NCH = BPW // C
SCOL = 32
VROWS = C * J


def _sc_body(gt_hbm, gidx_hbm, pu_hbm, pv_hbm, u_hbm, v_hbm, out_hbm,
             gt_v, gidx_v, pu_v, pv_v, urows_v, vrows_v, scores_v, sem):
    wid = lax.axis_index("s") * NC + lax.axis_index("c")
    base = wid * BPW
    pltpu.sync_copy(gt_hbm.at[pl.ds(base, BPW)], gt_v)
    pltpu.sync_copy(pu_hbm.at[pl.ds(base, BPW)], pu_v)
    pltpu.sync_copy(gidx_hbm.at[pl.ds(base * J, BPW * J)], gidx_v)
    pltpu.sync_copy(pv_hbm.at[pl.ds(base * J, BPW * J)], pv_v)

    lanes = lax.iota(jnp.int32, 16)
    m15 = lanes == 15
    zero = lanes * 0

    def cbody(c, carry):
        roff = pl.multiple_of(c * VROWS, 8)
        cps = [pltpu.async_copy(
            u_hbm.at[gt_v.at[pl.ds(pl.multiple_of(c * C, 8), C)]],
            urows_v, sem)]
        nfull, tail = VROWS // 128, VROWS % 128
        for i in range(nfull):
            cps.append(pltpu.async_copy(
                v_hbm.at[gidx_v.at[pl.ds(roff + i * 128, 128)]],
                vrows_v.at[pl.ds(i * 128, 128)], sem))
        if tail:
            cps.append(pltpu.async_copy(
                v_hbm.at[gidx_v.at[pl.ds(roff + nfull * 128, tail)]],
                vrows_v.at[pl.ds(nfull * 128, tail)], sem))
        for cp in cps:
            cp.wait()

        @plsc.parallel_loop(0, C, unroll=2)
        def bbody(b):
            pu_b = plsc.load_gather(pu_v, [zero + (c * C + b)])
            u = [jnp.where(pu_b == 1,
                           urows_v[b, pl.ds(64 + 16 * q, 16)],
                           urows_v[b, pl.ds(16 * q, 16)])
                 for q in range(4)]
            r0 = b * J
            bfull = zero + b
            f0 = zero + (c * C + b) * J
            for j in range(J):
                pv_bj = plsc.load_gather(pv_v, [f0 + j])
                p = zero.astype(jnp.float32)
                for q in range(4):
                    rsel = jnp.where(pv_bj == 1,
                                     vrows_v[r0 + j, pl.ds(64 + 16 * q, 16)],
                                     vrows_v[r0 + j, pl.ds(16 * q, 16)])
                    p = p + u[q] * rsel
                cs = plsc.cumsum(p)
                plsc.store_scatter(scores_v, [bfull, zero + j], cs,
                                   mask=m15)

        pltpu.sync_copy(
            scores_v,
            out_hbm.at[pl.ds(pl.multiple_of(base + c * C, 8), C), :])
        return carry

    lax.fori_loop(0, NCH, cbody, 0)


_sc_scores = functools.partial(
    pl.kernel, _sc_body,
    out_type=jax.ShapeDtypeStruct((B, SCOL), jnp.float32),
    mesh=plsc.VectorSubcoreMesh(core_axis_name="c", subcore_axis_name="s",
                                num_cores=NC, num_subcores=NS),
    compiler_params=pltpu.CompilerParams(needs_layout_passes=False),
    scratch_types=[
        pltpu.VMEM((BPW,), jnp.int32),
        pltpu.VMEM((BPW * J,), jnp.int32),
        pltpu.VMEM((BPW,), jnp.int32),
        pltpu.VMEM((BPW * J,), jnp.int32),
        pltpu.VMEM((C, 2 * D), jnp.float32),
        pltpu.VMEM((VROWS, 2 * D), jnp.float32),
        pltpu.VMEM((C, SCOL), jnp.float32),
        pltpu.SemaphoreType.DMA,
    ],
)()


EMB = 1000000
EMBP = 1000064
TJ = 512
NBF = EMB // TJ
GMAX = NBF // NW + 1


def _sc_repack_body(ut_hbm, vt_hbm, ud_hbm, vd_hbm, in_v, out_v, sem):
    wid = lax.axis_index("s") * NC + lax.axis_index("c")
    lanes = lax.iota(jnp.int32, 16)
    zero = lanes * 0
    rowidx = [lanes + 16 * q for q in range(4)]

    def transpose_block(src_hbm, dst_hbm, j0, length):
        j0 = pl.multiple_of(j0, 128)
        cps = [pltpu.async_copy(
            src_hbm.at[pl.ds(8 * r, 8), pl.ds(j0, length)],
            in_v.at[pl.ds(8 * r, 8), pl.ds(0, length)], sem)
            for r in range(8)]
        for cp in cps:
            cp.wait()

        @plsc.parallel_loop(0, length // 2, unroll=4)
        def pbody(p):
            for half in range(2):
                col = zero + (2 * p + half)
                for q in range(4):
                    out_v[p, pl.ds(64 * half + 16 * q, 16)] = (
                        plsc.load_gather(in_v, [rowidx[q], col]))

        pltpu.sync_copy(out_v.at[pl.ds(0, length // 2), :],
                        dst_hbm.at[pl.ds(pl.multiple_of(j0 // 2, 64),
                                         length // 2), :])

    def gbody(g, carry):
        blk = g * NW + wid

        @pl.when(blk < NBF)
        def _():
            transpose_block(ut_hbm, ud_hbm, blk * TJ, TJ)
            transpose_block(vt_hbm, vd_hbm, blk * TJ, TJ)

        return carry

    lax.fori_loop(0, GMAX, gbody, 0)

    @pl.when(wid == NBF % NW)
    def _():
        j0t = NBF * TJ + wid * 0
        transpose_block(ut_hbm, ud_hbm, j0t, 128)
        transpose_block(vt_hbm, vd_hbm, j0t, 128)


_sc_repack = functools.partial(
    pl.kernel, _sc_repack_body,
    out_type=(jax.ShapeDtypeStruct((EMBP // 2, 2 * D), jnp.float32),
              jax.ShapeDtypeStruct((EMBP // 2, 2 * D), jnp.float32)),
    mesh=plsc.VectorSubcoreMesh(core_axis_name="c", subcore_axis_name="s",
                                num_cores=NC, num_subcores=NS),
    compiler_params=pltpu.CompilerParams(needs_layout_passes=False),
    scratch_types=[
        pltpu.VMEM((D, TJ), jnp.float32),
        pltpu.VMEM((TJ // 2, 2 * D), jnp.float32),
        pltpu.SemaphoreType.DMA,
    ],
)()


def _tc_finish_body(s_ref, o_ref):
    x = s_ref[...]
    col = lax.broadcasted_iota(jnp.int32, x.shape, 1)
    y = jnp.where(col == 0, -x, x)
    sp = jnp.maximum(y, 0.0) + jnp.log1p(jnp.exp(-jnp.abs(y)))
    sp = jnp.where(col < J, sp, 0.0)
    o_ref[...] = jnp.sum(sp, axis=1)


_TCR = 2048


def _tc_finish(scores):
    return pl.pallas_call(
        _tc_finish_body,
        grid=(B // _TCR,),
        in_specs=[pl.BlockSpec((_TCR, SCOL), lambda i: (i, 0))],
        out_specs=pl.BlockSpec((_TCR,), lambda i: (i,)),
        out_shape=jax.ShapeDtypeStruct((B,), jnp.float32),
    )(scores)


def kernel(target, context, neg, u_weight, v_weight):
    tgt = target.astype(jnp.int32)
    cat = jnp.concatenate(
        [context.astype(jnp.int32)[:, None], neg.astype(jnp.int32)],
        axis=1).reshape(-1)
    u2, v2 = _sc_repack(u_weight.T, v_weight.T)
    scores = _sc_scores(tgt >> 1, cat >> 1, tgt & 1, cat & 1, u2, v2)
    return _tc_finish(scores)


# speedup vs baseline: 3.2211x; 1.9392x over previous
import functools

import jax
import jax.numpy as jnp
from jax import lax
from jax.experimental import pallas as pl
from jax.experimental.pallas import tpu as pltpu
from jax.experimental.pallas import tpu_sc as plsc

NC = 2
NS = 16
NW = NC * NS

B = 16384
D = 64
K = 20
J = K + 1
BPW = B // NW
C = 32
NCH = BPW // C
SCOL = 32
VROWS = C * J


def _sc_body(tgt_hbm, cat_hbm, u_hbm, v_hbm, out_hbm,
             tgt_v, idx_v, urows_v, vrows_v, scores_v, sem0, sem1):
    wid = lax.axis_index("s") * NC + lax.axis_index("c")
    base = wid * BPW
    pltpu.sync_copy(tgt_hbm.at[pl.ds(base, BPW)], tgt_v)
    pltpu.sync_copy(cat_hbm.at[pl.ds(base * J, BPW * J)], idx_v)

    lanes = lax.iota(jnp.int32, 16)
    m15 = lanes == 15
    zero = lanes * 0

    def chunk_copies(c, buf, sem):
        roff = pl.multiple_of(c * VROWS, 8)
        cps = [pltpu.make_async_copy(
            u_hbm.at[tgt_v.at[pl.ds(pl.multiple_of(c * C, 8), C)]],
            urows_v.at[buf], sem)]
        nfull, tail = VROWS // 128, VROWS % 128
        for i in range(nfull):
            cps.append(pltpu.make_async_copy(
                v_hbm.at[idx_v.at[pl.ds(roff + i * 128, 128)]],
                vrows_v.at[buf, pl.ds(i * 128, 128)], sem))
        if tail:
            cps.append(pltpu.make_async_copy(
                v_hbm.at[idx_v.at[pl.ds(roff + nfull * 128, tail)]],
                vrows_v.at[buf, pl.ds(nfull * 128, tail)], sem))
        return cps

    def fire(c, buf, sem):
        for cp in chunk_copies(c, buf, sem):
            cp.start()

    def drain(c, buf, sem):
        for cp in chunk_copies(c, buf, sem):
            cp.wait()

    def compute_chunk(c, buf):
        @plsc.parallel_loop(0, C, unroll=2)
        def bbody(b):
            u0 = urows_v[buf, b, pl.ds(0, 16)]
            u1 = urows_v[buf, b, pl.ds(16, 16)]
            u2 = urows_v[buf, b, pl.ds(32, 16)]
            u3 = urows_v[buf, b, pl.ds(48, 16)]
            r0 = b * J
            bfull = zero + b
            for j in range(J):
                p = (u0 * vrows_v[buf, r0 + j, pl.ds(0, 16)]
                     + u1 * vrows_v[buf, r0 + j, pl.ds(16, 16)]
                     + u2 * vrows_v[buf, r0 + j, pl.ds(32, 16)]
                     + u3 * vrows_v[buf, r0 + j, pl.ds(48, 16)])
                cs = plsc.cumsum(p)
                plsc.store_scatter(scores_v, [bfull, zero + j], cs,
                                   mask=m15)

        pltpu.sync_copy(
            scores_v,
            out_hbm.at[pl.ds(pl.multiple_of(base + c * C, 8), C), :])

    fire(0, 0, sem0)

    def cbody(g, carry):
        c = g * 2
        fire(c + 1, 1, sem1)
        drain(c, 0, sem0)
        compute_chunk(c, 0)

        @pl.when(c + 2 < NCH)
        def _():
            fire(c + 2, 0, sem0)

        drain(c + 1, 1, sem1)
        compute_chunk(c + 1, 1)
        return carry

    lax.fori_loop(0, NCH // 2, cbody, 0)


_sc_scores = functools.partial(
    pl.kernel, _sc_body,
    out_type=jax.ShapeDtypeStruct((B, SCOL), jnp.float32),
    mesh=plsc.VectorSubcoreMesh(core_axis_name="c", subcore_axis_name="s",
                                num_cores=NC, num_subcores=NS),
    compiler_params=pltpu.CompilerParams(needs_layout_passes=False,
                                         use_tc_tiling_on_sc=False),
    scratch_types=[
        pltpu.VMEM((BPW,), jnp.int32),
        pltpu.VMEM((BPW * J,), jnp.int32),
        pltpu.VMEM((2, C, D), jnp.float32),
        pltpu.VMEM((2, VROWS, D), jnp.float32),
        pltpu.VMEM((C, SCOL), jnp.float32),
        pltpu.SemaphoreType.DMA,
        pltpu.SemaphoreType.DMA,
    ],
)()


def _tc_finish_body(s_ref, o_ref):
    x = s_ref[...]
    col = lax.broadcasted_iota(jnp.int32, x.shape, 1)
    y = jnp.where(col == 0, -x, x)
    sp = jnp.maximum(y, 0.0) + jnp.log1p(jnp.exp(-jnp.abs(y)))
    sp = jnp.where(col < J, sp, 0.0)
    o_ref[...] = jnp.sum(sp, axis=1)


_TCR = 2048


def _tc_finish(scores):
    return pl.pallas_call(
        _tc_finish_body,
        grid=(B // _TCR,),
        in_specs=[pl.BlockSpec((_TCR, SCOL), lambda i: (i, 0))],
        out_specs=pl.BlockSpec((_TCR,), lambda i: (i,)),
        out_shape=jax.ShapeDtypeStruct((B,), jnp.float32),
    )(scores)


def kernel(target, context, neg, u_weight, v_weight):
    tgt = target.astype(jnp.int32)
    cat = jnp.concatenate(
        [context.astype(jnp.int32)[:, None], neg.astype(jnp.int32)],
        axis=1).reshape(-1)
    scores = _sc_scores(tgt, cat, u_weight, v_weight)
    return _tc_finish(scores)
